# baseline (speedup 1.0000x reference)
import jax
import jax.numpy as jnp
from jax.experimental import pallas as pl

B = 1024
FEAT = 840
LATENT = 128
HIDDEN = 256
N_Q = 12
N_EMB = 512


def _rowsum_xla_order(s):
    t = s[:, 0:8]
    for g in range(1, 16):
        t = t + s[:, 8 * g:8 * g + 8]
    w = 4
    while w >= 1:
        t = t[:, :w] + t[:, w:2 * w]
        w //= 2
    return t


def _rvq_body(z_ref, cb_ref, cbsq_ref, q_ref, i_ref):
    residual = z_ref[:]
    quant = jnp.zeros_like(residual)
    iota = jax.lax.broadcasted_iota(jnp.int32, (B, N_EMB), 1)
    for i in range(N_Q):
        cb = cb_ref[i]
        rsq = _rowsum_xla_order(residual * residual)
        mm = jax.lax.dot_general(
            residual, cb, (((1,), (1,)), ((), ())),
            preferred_element_type=jnp.float32)
        dist = (rsq - 2.0 * mm) + cbsq_ref[i][None, :]
        m = jnp.min(dist, axis=1, keepdims=True)
        idx = jnp.min(jnp.where(dist == m, iota, N_EMB), axis=1)
        i_ref[i, :] = idx
        oh = (iota == idx[:, None]).astype(jnp.float32)
        qv = jax.lax.dot_general(
            oh, cb, (((1,), (0,)), ((), ())),
            precision=jax.lax.Precision.HIGHEST,
            preferred_element_type=jnp.float32)
        quant = quant + qv
        residual = residual - qv
    q_ref[:] = quant


def kernel(x, conv1_w, conv1_b, conv2_w, conv2_b, codebooks):
    h = x[:, None, :]
    h = jax.nn.relu(jax.lax.conv_general_dilated(
        h, conv1_w, window_strides=(1,), padding=((1, 1),),
        dimension_numbers=("NCH", "OIH", "NCH")) + conv1_b[None, :, None])
    h = jax.nn.relu(jax.lax.conv_general_dilated(
        h, conv2_w, window_strides=(1,), padding=((1, 1),),
        dimension_numbers=("NCH", "OIH", "NCH")) + conv2_b[None, :, None])
    z = jnp.mean(h, axis=2)

    cbsq = jnp.stack([jnp.sum(codebooks[i] ** 2, axis=1) for i in range(N_Q)],
                     axis=0)

    quant, idx_t = pl.pallas_call(
        _rvq_body,
        out_shape=(jax.ShapeDtypeStruct((B, LATENT), jnp.float32),
                   jax.ShapeDtypeStruct((N_Q, B), jnp.int32)),
    )(z, codebooks, cbsq)
    zq = quant.reshape(B, 1, LATENT)
    indices = idx_t.T.reshape(B, 1, N_Q)
    return (zq, indices)


# speedup vs baseline: 1.0631x; 1.0631x over previous
import jax
import jax.numpy as jnp
from jax.experimental import pallas as pl

B = 1024
FEAT = 840
LATENT = 128
HIDDEN = 256
N_Q = 12
N_EMB = 512


def _rowsum_xla_order(s):
    t = s[:, 0:8]
    for g in range(1, 16):
        t = t + s[:, 8 * g:8 * g + 8]
    w = 4
    while w >= 1:
        t = t[:, :w] + t[:, w:2 * w]
        w //= 2
    return t


def _rvq_body(z_ref, cb_ref, cbsq_ref, hi_ref, mid_ref, lo_ref, q_ref, i_ref):
    residual = z_ref[:]
    quant = jnp.zeros_like(residual)
    iota = jax.lax.broadcasted_iota(jnp.int32, (B, N_EMB), 1)
    for i in range(N_Q):
        cb = cb_ref[i]
        rsq = _rowsum_xla_order(residual * residual)
        mm = jax.lax.dot_general(
            residual, cb, (((1,), (1,)), ((), ())),
            preferred_element_type=jnp.float32)
        dist = (rsq - 2.0 * mm) + cbsq_ref[i][None, :]
        m = jnp.min(dist, axis=1, keepdims=True)
        idx = jnp.min(jnp.where(dist == m, iota, N_EMB), axis=1)
        i_ref[i, :] = idx
        oh = (iota == idx[:, None]).astype(jnp.float32)
        def _sel(a_ref):
            return jax.lax.dot_general(
                oh, a_ref[i], (((1,), (0,)), ((), ())),
                preferred_element_type=jnp.float32)
        qv = (_sel(hi_ref) + _sel(mid_ref)) + _sel(lo_ref)
        quant = quant + qv
        residual = residual - qv
    q_ref[:] = quant


def kernel(x, conv1_w, conv1_b, conv2_w, conv2_b, codebooks):
    h = x[:, None, :]
    h = jax.nn.relu(jax.lax.conv_general_dilated(
        h, conv1_w, window_strides=(1,), padding=((1, 1),),
        dimension_numbers=("NCH", "OIH", "NCH")) + conv1_b[None, :, None])
    h = jax.nn.relu(jax.lax.conv_general_dilated(
        h, conv2_w, window_strides=(1,), padding=((1, 1),),
        dimension_numbers=("NCH", "OIH", "NCH")) + conv2_b[None, :, None])
    z = jnp.mean(h, axis=2)

    cbsq = jnp.stack([jnp.sum(codebooks[i] ** 2, axis=1) for i in range(N_Q)],
                     axis=0)
    cb_hi = codebooks.astype(jnp.bfloat16).astype(jnp.float32)
    cb_mid = (codebooks - cb_hi).astype(jnp.bfloat16).astype(jnp.float32)
    cb_lo = (codebooks - cb_hi) - cb_mid

    quant, idx_t = pl.pallas_call(
        _rvq_body,
        out_shape=(jax.ShapeDtypeStruct((B, LATENT), jnp.float32),
                   jax.ShapeDtypeStruct((N_Q, B), jnp.int32)),
    )(z, codebooks, cbsq, cb_hi, cb_mid, cb_lo)
    zq = quant.reshape(B, 1, LATENT)
    indices = idx_t.T.reshape(B, 1, N_Q)
    return (zq, indices)
